# baseline (speedup 1.0000x reference)
import functools

import jax
import jax.numpy as jnp
from jax import lax
from jax.experimental import pallas as pl
from jax.experimental.pallas import tpu as pltpu
from jax.experimental.pallas import tpu_sc as plsc

B = 16384
E_INV = 64
E_TK = 32
E_DT = 32
NC = 2
NS = 16
NW = NC * NS
BPW = B // NW
CHUNK = 128
NCH = BPW // CHUNK

_mesh = plsc.VectorSubcoreMesh(core_axis_name="c", subcore_axis_name="s")


@functools.partial(
    pl.kernel,
    out_type=jax.ShapeDtypeStruct((B,), jnp.float32),
    mesh=_mesh,
    compiler_params=pltpu.CompilerParams(
        needs_layout_passes=False, use_tc_tiling_on_sc=False),
    scratch_types=[
        pltpu.VMEM((BPW,), jnp.int32),
        pltpu.VMEM((BPW,), jnp.int32),
        pltpu.VMEM((BPW,), jnp.int32),
        pltpu.VMEM((BPW, E_INV), jnp.float32),
        pltpu.VMEM((BPW, E_TK), jnp.float32),
        pltpu.VMEM((BPW, E_DT), jnp.float32),
        pltpu.VMEM((BPW,), jnp.float32),
        pltpu.SemaphoreType.DMA,
    ],
)
def _mf_kernel(inv_ids_hbm, tk_ids_hbm, dt_ids_hbm, wi_hbm, wt_hbm, wd_hbm,
               out_hbm, inv_idx, tk_idx, dt_idx, inv_rows, tk_rows, dt_rows,
               out_v, sem):
    wid = lax.axis_index("s") * NC + lax.axis_index("c")
    base = wid * BPW

    pltpu.sync_copy(inv_ids_hbm.at[pl.ds(base, BPW)], inv_idx)
    pltpu.sync_copy(tk_ids_hbm.at[pl.ds(base, BPW)], tk_idx)
    pltpu.sync_copy(dt_ids_hbm.at[pl.ds(base, BPW)], dt_idx)

    copies = []
    for j in range(NCH):
        s = j * CHUNK
        copies.append(pltpu.async_copy(
            wi_hbm.at[inv_idx.at[pl.ds(s, CHUNK)]],
            inv_rows.at[pl.ds(s, CHUNK)], sem))
        copies.append(pltpu.async_copy(
            wt_hbm.at[tk_idx.at[pl.ds(s, CHUNK)]],
            tk_rows.at[pl.ds(s, CHUNK)], sem))
        copies.append(pltpu.async_copy(
            wd_hbm.at[dt_idx.at[pl.ds(s, CHUNK)]],
            dt_rows.at[pl.ds(s, CHUNK)], sem))
    for c in copies:
        c.wait()

    iota = lax.iota(jnp.int32, 16)

    def body(c, carry):
        b0 = c * 16
        rows = b0 + iota
        acc = jnp.zeros((16,), jnp.float32)
        for d in range(E_TK):
            dv = jnp.full((16,), d, jnp.int32)
            dv2 = jnp.full((16,), d + E_TK, jnp.int32)
            acc += plsc.load_gather(inv_rows, [rows, dv]) * \
                plsc.load_gather(tk_rows, [rows, dv])
            acc += plsc.load_gather(inv_rows, [rows, dv2]) * \
                plsc.load_gather(dt_rows, [rows, dv])
        out_v[pl.ds(b0, 16)] = acc
        return carry

    lax.fori_loop(0, BPW // 16, body, 0)

    pltpu.sync_copy(out_v, out_hbm.at[pl.ds(base, BPW)])


def kernel(investor_ids, ticker_ids, date_ids, W_investor, W_ticker, W_date):
    return _mf_kernel(investor_ids, ticker_ids, date_ids,
                      W_investor, W_ticker, W_date)


# speedup vs baseline: 1.0287x; 1.0287x over previous
import functools

import jax
import jax.numpy as jnp
from jax import lax
from jax.experimental import pallas as pl
from jax.experimental.pallas import tpu as pltpu
from jax.experimental.pallas import tpu_sc as plsc

B = 16384
N_INV = 1000000
N_TK = 100000
N_DT = 1000
NC = 2
NS = 16
NW = NC * NS
BPW = B // NW
CH = 128
NCH = BPW // CH
L = 16

_mesh = plsc.VectorSubcoreMesh(core_axis_name="c", subcore_axis_name="s")


@functools.partial(
    pl.kernel,
    out_type=jax.ShapeDtypeStruct((B,), jnp.float32),
    mesh=_mesh,
    compiler_params=pltpu.CompilerParams(
        needs_layout_passes=False, use_tc_tiling_on_sc=True),
    scratch_types=[
        pltpu.VMEM((BPW,), jnp.int32),
        pltpu.VMEM((BPW,), jnp.int32),
        pltpu.VMEM((BPW,), jnp.int32),
        pltpu.VMEM((BPW,), jnp.int32),
        pltpu.VMEM((BPW,), jnp.int32),
        pltpu.VMEM((BPW,), jnp.int32),
        pltpu.VMEM((BPW,), jnp.int32),
        pltpu.VMEM((BPW,), jnp.int32),
        pltpu.VMEM((BPW,), jnp.int32),
        pltpu.VMEM((2, CH, 128), jnp.float32),
        pltpu.VMEM((2, CH, 128), jnp.float32),
        pltpu.VMEM((2, CH, 128), jnp.float32),
        pltpu.VMEM((BPW + L,), jnp.float32),
        pltpu.SemaphoreType.DMA,
        pltpu.SemaphoreType.DMA,
    ],
)
def _mf_kernel(inv_ids_hbm, tk_ids_hbm, dt_ids_hbm, wi_hbm, wt_hbm, wd_hbm,
               out_hbm, inv_ids, tk_ids, dt_ids, inv_ti, tk_ti, dt_ti,
               bi_v, bt_v, bd_v, inv_buf, tk_buf, dt_buf, out_v, sem0, sem1):
    wid = lax.axis_index("s") * NC + lax.axis_index("c")
    base = wid * BPW
    sems = (sem0, sem1)

    pltpu.sync_copy(inv_ids_hbm.at[pl.ds(base, BPW)], inv_ids)
    pltpu.sync_copy(tk_ids_hbm.at[pl.ds(base, BPW)], tk_ids)
    pltpu.sync_copy(dt_ids_hbm.at[pl.ds(base, BPW)], dt_ids)

    def prep(g, carry):
        s = pl.ds(g * L, L)
        iv = inv_ids[s]
        inv_ti[s] = iv >> 1
        bi_v[s] = (iv & 1) * 64
        tv = tk_ids[s]
        tk_ti[s] = tv >> 2
        bt_v[s] = (tv & 3) * 32
        dv = dt_ids[s]
        dt_ti[s] = dv >> 2
        bd_v[s] = (dv & 3) * 32
        return carry

    lax.fori_loop(0, BPW // L, prep, 0)

    def fire(c, bank):
        s = pl.ds(c * CH, CH)
        return [
            pltpu.async_copy(wi_hbm.at[inv_ti.at[s]], inv_buf.at[bank], sems[bank]),
            pltpu.async_copy(wt_hbm.at[tk_ti.at[s]], tk_buf.at[bank], sems[bank]),
            pltpu.async_copy(wd_hbm.at[dt_ti.at[s]], dt_buf.at[bank], sems[bank]),
        ]

    lastmask = lax.iota(jnp.int32, L) == (L - 1)
    inflight = fire(0, 0)
    for c in range(NCH):
        bank = c & 1
        pending = inflight
        if c + 1 < NCH:
            inflight = fire(c + 1, bank ^ 1)
        for cp in pending:
            cp.wait()

        def group(g, carry, *, bank=bank, c=c):
            gbase = c * CH + g * L
            sg = pl.ds(gbase, L)
            biv = bi_v[sg]
            btv = bt_v[sg]
            bdv = bd_v[sg]
            for j in range(L):
                r = g * L + j
                bi = biv[j]
                bt = btv[j]
                bd = bdv[j]
                a = inv_buf[bank, r, pl.ds(bi, L)] * tk_buf[bank, r, pl.ds(bt, L)]
                a += inv_buf[bank, r, pl.ds(bi + 16, L)] * tk_buf[bank, r, pl.ds(bt + 16, L)]
                a += inv_buf[bank, r, pl.ds(bi + 32, L)] * dt_buf[bank, r, pl.ds(bd, L)]
                a += inv_buf[bank, r, pl.ds(bi + 48, L)] * dt_buf[bank, r, pl.ds(bd + 16, L)]
                cs = plsc.cumsum(a)
                plsc.store_compressed(out_v.at[pl.ds(gbase + j, L)], cs,
                                      mask=lastmask)
            return carry

        lax.fori_loop(0, CH // L, group, 0)

    pltpu.sync_copy(out_v.at[pl.ds(0, BPW)], out_hbm.at[pl.ds(base, BPW)])


def kernel(investor_ids, ticker_ids, date_ids, W_investor, W_ticker, W_date):
    wi2 = W_investor.reshape(N_INV // 2, 128)
    wt2 = W_ticker.reshape(N_TK // 4, 128)
    wd2 = W_date.reshape(N_DT // 4, 128)
    return _mf_kernel(investor_ids, ticker_ids, date_ids, wi2, wt2, wd2)


# speedup vs baseline: 1.1315x; 1.1000x over previous
import functools

import jax
import jax.numpy as jnp
from jax import lax
from jax.experimental import pallas as pl
from jax.experimental.pallas import tpu as pltpu
from jax.experimental.pallas import tpu_sc as plsc

B = 16384
NC = 2
NS = 16
NW = NC * NS
BPW = B // NW
CH = 128
NCH = BPW // CH
L = 16

_mesh = plsc.VectorSubcoreMesh(core_axis_name="c", subcore_axis_name="s")


@functools.partial(
    pl.kernel,
    out_type=jax.ShapeDtypeStruct((B,), jnp.float32),
    mesh=_mesh,
    compiler_params=pltpu.CompilerParams(
        needs_layout_passes=False, use_tc_tiling_on_sc=True),
    scratch_types=[
        pltpu.VMEM((BPW,), jnp.int32),
        pltpu.VMEM((BPW,), jnp.int32),
        pltpu.VMEM((BPW,), jnp.int32),
        pltpu.VMEM((2, CH, 128), jnp.float32),
        pltpu.VMEM((2, CH, 128), jnp.float32),
        pltpu.VMEM((2, CH, 128), jnp.float32),
        pltpu.VMEM((BPW + L,), jnp.float32),
        pltpu.SemaphoreType.DMA,
        pltpu.SemaphoreType.DMA,
    ],
)
def _mf_kernel(inv_ids_hbm, tk_ids_hbm, dt_ids_hbm, wi_hbm, wt_hbm, wd_hbm,
               out_hbm, inv_ids, tk_ids, dt_ids, inv_buf, tk_buf, dt_buf,
               out_v, sem0, sem1):
    wid = lax.axis_index("s") * NC + lax.axis_index("c")
    base = wid * BPW
    sems = (sem0, sem1)

    pltpu.sync_copy(inv_ids_hbm.at[pl.ds(base, BPW)], inv_ids)
    pltpu.sync_copy(tk_ids_hbm.at[pl.ds(base, BPW)], tk_ids)
    pltpu.sync_copy(dt_ids_hbm.at[pl.ds(base, BPW)], dt_ids)

    def fire(c, bank):
        s = pl.ds(c * CH, CH)
        return [
            pltpu.async_copy(wi_hbm.at[inv_ids.at[s]], inv_buf.at[bank], sems[bank]),
            pltpu.async_copy(wt_hbm.at[tk_ids.at[s]], tk_buf.at[bank], sems[bank]),
            pltpu.async_copy(wd_hbm.at[dt_ids.at[s]], dt_buf.at[bank], sems[bank]),
        ]

    lastmask = lax.iota(jnp.int32, L) == (L - 1)
    inflight = fire(0, 0)
    for c in range(NCH):
        bank = c & 1
        pending = inflight
        if c + 1 < NCH:
            inflight = fire(c + 1, bank ^ 1)
        for cp in pending:
            cp.wait()

        def group(g, carry, *, bank=bank, c=c):
            gbase = c * CH + g * L
            for j in range(L):
                r = g * L + j
                a = inv_buf[bank, r, pl.ds(0, L)] * tk_buf[bank, r, pl.ds(0, L)]
                a += inv_buf[bank, r, pl.ds(16, L)] * tk_buf[bank, r, pl.ds(16, L)]
                a += inv_buf[bank, r, pl.ds(32, L)] * dt_buf[bank, r, pl.ds(0, L)]
                a += inv_buf[bank, r, pl.ds(48, L)] * dt_buf[bank, r, pl.ds(16, L)]
                cs = plsc.cumsum(a)
                plsc.store_compressed(out_v.at[pl.ds(gbase + j, L)], cs,
                                      mask=lastmask)
            return carry

        lax.fori_loop(0, CH // L, group, 0)

    pltpu.sync_copy(out_v.at[pl.ds(0, BPW)], out_hbm.at[pl.ds(base, BPW)])


def kernel(investor_ids, ticker_ids, date_ids, W_investor, W_ticker, W_date):
    wi_p = jnp.pad(W_investor, ((0, 0), (0, 64)))
    wt_p = jnp.pad(W_ticker, ((0, 0), (0, 96)))
    wd_p = jnp.pad(W_date, ((0, 0), (0, 96)))
    return _mf_kernel(investor_ids, ticker_ids, date_ids, wi_p, wt_p, wd_p)


# speedup vs baseline: 1.7623x; 1.5574x over previous
import functools

import jax
import jax.numpy as jnp
from jax import lax
from jax.experimental import pallas as pl
from jax.experimental.pallas import tpu as pltpu
from jax.experimental.pallas import tpu_sc as plsc

B = 16384
N_INV = 1000000
NC = 2
NS = 16
NW = NC * NS
BPW = B // NW
L = 16

RANGE = 31232
CW = 512
NCHUNK = RANGE // CW
HCAP = 1024
BCAP = 48
SCAP = 32
PITCH = 136
NDUMP = 64

_mesh = plsc.VectorSubcoreMesh(core_axis_name="c", subcore_axis_name="s")
_params = pltpu.CompilerParams(
    needs_layout_passes=False, use_tc_tiling_on_sc=True)


@functools.partial(
    pl.kernel,
    out_type=jax.ShapeDtypeStruct((B + NDUMP, 128), jnp.float32),
    mesh=_mesh,
    compiler_params=_params,
    scratch_types=[
        pltpu.VMEM((B,), jnp.int32),
        pltpu.VMEM((HCAP + L,), jnp.int32),
        pltpu.VMEM((HCAP + L,), jnp.int32),
        pltpu.VMEM((64, BCAP + L), jnp.int32),
        pltpu.VMEM((64, BCAP + L), jnp.int32),
        pltpu.VMEM((64 + L,), jnp.int32),
        pltpu.VMEM((64, CW), jnp.float32),
        pltpu.VMEM((64, 64), jnp.float32),
        pltpu.VMEM((SCAP, PITCH), jnp.float32),
        pltpu.VMEM((8, SCAP), jnp.int32),
    ],
)
def _extract_kernel(inv_ids_hbm, wi_t, i1_hbm, aids, qlist, plist,
                    bq, bp, bcnt, chunk, tailbuf, stg, idxb):
    wid = lax.axis_index("s") * NC + lax.axis_index("c")
    lo = wid * RANGE
    is_last = wid == NW - 1
    hi = jnp.where(is_last, N_INV, lo + RANGE)
    iota = lax.iota(jnp.int32, L)

    pltpu.sync_copy(inv_ids_hbm, aids)

    def scan(g, cnt):
        v = aids[pl.ds(g * L, L)]
        m = (v >= lo) & (v < hi)
        plsc.store_compressed(qlist.at[pl.ds(cnt, L)], v - lo, mask=m)
        plsc.store_compressed(plist.at[pl.ds(cnt, L)], g * L + iota, mask=m)
        return cnt + plsc.all_reduce_population_count(m)[0]

    cnt = lax.fori_loop(0, B // L, scan, 0)
    ngrp = (cnt + L - 1) // L

    def bucket(bk, carry):
        def fill(g, cb):
            qv = qlist[pl.ds(g * L, L)]
            valid = (g * L + iota) < cnt
            m = ((qv >> 9) == bk) & valid
            plsc.store_compressed(bq.at[bk, pl.ds(cb, L)], qv & (CW - 1),
                                  mask=m)
            plsc.store_compressed(bp.at[bk, pl.ds(cb, L)],
                                  plist[pl.ds(g * L, L)], mask=m)
            return cb + plsc.all_reduce_population_count(m)[0]
        cb = lax.fori_loop(0, ngrp, fill, 0)
        plsc.store_compressed(bcnt.at[pl.ds(bk, L)], jnp.full((L,), cb),
                              mask=iota == 0)
        return carry

    lax.fori_loop(0, 64, bucket, 0)

    def extract(bk, src, qm):
        nb = bcnt[pl.ds(bk, L)][0]
        for hb in range(SCAP // L):
            sl = pl.ds(hb * L, L)
            qv = bq[bk, sl] & qm
            pv = bp[bk, sl]
            m = (hb * L + iota) < nb
            for d in range(64):
                val = plsc.load_gather(src, [jnp.full((L,), d), qv])
                plsc.store_scatter(stg, [hb * L + iota, jnp.full((L,), d)],
                                   val, mask=m)
            idxb[0, pl.ds(hb * L, L)] = jnp.where(m, pv, B + hb * L + iota)
        pltpu.sync_copy(stg.at[:, pl.ds(0, 128)], i1_hbm.at[idxb.at[0]])

    def stream(k, carry):
        pltpu.sync_copy(wi_t.at[:, pl.ds(lo + k * CW, CW)], chunk)
        extract(k, chunk, CW - 1)
        return carry

    nfull = jnp.where(is_last, NCHUNK + 1, NCHUNK)
    lax.fori_loop(0, nfull, stream, 0)

    @pl.when(is_last)
    def _tail():
        pltpu.sync_copy(wi_t.at[:, pl.ds(N_INV - 64, 64)], tailbuf)
        extract(NCHUNK + 1, tailbuf, 63)


@functools.partial(
    pl.kernel,
    out_type=jax.ShapeDtypeStruct((B,), jnp.float32),
    mesh=_mesh,
    compiler_params=_params,
    scratch_types=[
        pltpu.VMEM((BPW,), jnp.int32),
        pltpu.VMEM((BPW,), jnp.int32),
        pltpu.VMEM((BPW,), jnp.int32),
        pltpu.VMEM((BPW,), jnp.int32),
        pltpu.VMEM((BPW,), jnp.int32),
        pltpu.VMEM((BPW,), jnp.int32),
        pltpu.VMEM((2, 128, 128), jnp.float32),
        pltpu.VMEM((2, 128, 128), jnp.float32),
        pltpu.VMEM((2, 128, 128), jnp.float32),
        pltpu.VMEM((BPW + L,), jnp.float32),
        pltpu.SemaphoreType.DMA,
        pltpu.SemaphoreType.DMA,
    ],
)
def _dot_kernel(tk_ids_hbm, dt_ids_hbm, i1_hbm, wt_hbm, wd_hbm, out_hbm,
                tk_ids, dt_ids, tk_ti, dt_ti, bt_v, bd_v,
                inv_buf, tk_buf, dt_buf, out_v, sem0, sem1):
    wid = lax.axis_index("s") * NC + lax.axis_index("c")
    base = wid * BPW
    sems = (sem0, sem1)
    CH = 128
    NCH = BPW // CH

    pltpu.sync_copy(tk_ids_hbm.at[pl.ds(base, BPW)], tk_ids)
    pltpu.sync_copy(dt_ids_hbm.at[pl.ds(base, BPW)], dt_ids)

    def prep(g, carry):
        s = pl.ds(g * L, L)
        tv = tk_ids[s]
        tk_ti[s] = tv >> 2
        bt_v[s] = (tv & 3) * 32
        dv = dt_ids[s]
        dt_ti[s] = dv >> 2
        bd_v[s] = (dv & 3) * 32
        return carry

    lax.fori_loop(0, BPW // L, prep, 0)

    def fire(c, bank):
        s = pl.ds(c * CH, CH)
        return [
            pltpu.async_copy(i1_hbm.at[pl.ds(base + c * CH, CH)],
                             inv_buf.at[bank], sems[bank]),
            pltpu.async_copy(wt_hbm.at[tk_ti.at[s]], tk_buf.at[bank],
                             sems[bank]),
            pltpu.async_copy(wd_hbm.at[dt_ti.at[s]], dt_buf.at[bank],
                             sems[bank]),
        ]

    lastmask = lax.iota(jnp.int32, L) == (L - 1)
    inflight = fire(0, 0)
    for c in range(NCH):
        bank = c & 1
        pending = inflight
        if c + 1 < NCH:
            inflight = fire(c + 1, bank ^ 1)
        for cp in pending:
            cp.wait()

        def group(g, carry, *, bank=bank, c=c):
            gbase = c * CH + g * L
            sg = pl.ds(gbase, L)
            btv = bt_v[sg]
            bdv = bd_v[sg]
            for j in range(L):
                r = g * L + j
                bt = btv[j]
                bd = bdv[j]
                a = inv_buf[bank, r, pl.ds(0, L)] * tk_buf[bank, r, pl.ds(bt, L)]
                a += inv_buf[bank, r, pl.ds(16, L)] * tk_buf[bank, r, pl.ds(bt + 16, L)]
                a += inv_buf[bank, r, pl.ds(32, L)] * dt_buf[bank, r, pl.ds(bd, L)]
                a += inv_buf[bank, r, pl.ds(48, L)] * dt_buf[bank, r, pl.ds(bd + 16, L)]
                cs = plsc.cumsum(a)
                plsc.store_compressed(out_v.at[pl.ds(gbase + j, L)], cs,
                                      mask=lastmask)
            return carry

        lax.fori_loop(0, CH // L, group, 0)

    pltpu.sync_copy(out_v.at[pl.ds(0, BPW)], out_hbm.at[pl.ds(base, BPW)])


def kernel(investor_ids, ticker_ids, date_ids, W_investor, W_ticker, W_date):
    i1 = _extract_kernel(investor_ids, W_investor.T)
    wt2 = W_ticker.reshape(25000, 128)
    wd2 = W_date.reshape(250, 128)
    return _dot_kernel(ticker_ids, date_ids, i1, wt2, wd2)
